# baseline (speedup 1.0000x reference)
import jax
import jax.numpy as jnp
from jax.experimental import pallas as pl
from jax.experimental.pallas import tpu as pltpu

_R, _C = 16384, 2048
_BLK = 512


def _tc_body(out_ref, lab_ref, sum_ref):
    x = out_ref[...]
    lab = lab_ref[...]
    loss = jnp.sqrt(x * x + 1.0) - 1.0
    eps = jnp.float32(1e-31)
    anorm = -jnp.log(jnp.maximum(1.0 - jnp.exp(-loss), eps))
    v = jnp.where(lab == 1, anorm, loss)
    part = jnp.sum(v)

    @pl.when(pl.program_id(0) == 0)
    def _():
        sum_ref[0, 0] = 0.0

    sum_ref[0, 0] += part


def kernel(out, label):
    grid = _R // _BLK
    total = pl.pallas_call(
        _tc_body,
        grid=(grid,),
        in_specs=[
            pl.BlockSpec((_BLK, _C), lambda i: (i, 0)),
            pl.BlockSpec((_BLK, _C), lambda i: (i, 0)),
        ],
        out_specs=pl.BlockSpec(memory_space=pltpu.SMEM),
        out_shape=jax.ShapeDtypeStruct((1, 1), jnp.float32),
    )(out, label)
    return total[0, 0] * (1.0 / (_R * _C))
